# baseline (speedup 1.0000x reference)
import functools

import jax
import jax.numpy as jnp
from jax import lax
from jax.experimental import pallas as pl
from jax.experimental.pallas import tpu as pltpu
from jax.experimental.pallas import tpu_sc as plsc

N = 10000
E = 320000
F = 128
H = 128
O = 64

NC = 2
NS = 16
NW = NC * NS
EPT = E // NW
K = 80
CHUNKS = EPT // K
NPAD = 10240
RPT = 632
NR = NS * RPT
DPT = NPAD // NS

_mesh = plsc.VectorSubcoreMesh(core_axis_name="c", subcore_axis_name="s")
_sc_params = pltpu.CompilerParams(use_tc_tiling_on_sc=False)


DEG_U = 5


def _deg_body(src2_hbm, dst2_hbm, out_hbm,
              srcall, dstall, ones_v, zv, dsrc_sh, ddst_sh, sem):
    c = lax.axis_index("c")
    s = lax.axis_index("s")
    wid = s * NC + c
    pltpu.async_copy(src2_hbm.at[wid], srcall, sem)
    pltpu.async_copy(dst2_hbm.at[wid], dstall, sem)
    for i in range(K // 16):
        ones_v[pl.ds(i * 16, 16)] = jnp.ones((16,), jnp.float32)

    def zfill(i, carry):
        zv[pl.ds(i * 16, 16)] = jnp.zeros((16,), jnp.float32)
        return carry

    lax.fori_loop(0, DPT // 16, zfill, 0)
    pltpu.sync_copy(zv, dsrc_sh.at[pl.ds(s * DPT, DPT)])
    pltpu.sync_copy(zv, ddst_sh.at[pl.ds(s * DPT, DPT)])
    pltpu.make_async_copy(src2_hbm.at[wid], srcall, sem).wait()
    pltpu.make_async_copy(dst2_hbm.at[wid], dstall, sem).wait()
    plsc.subcore_barrier()

    def body(t, carry):
        for u in range(DEG_U):
            j = t * DEG_U + u
            pltpu.async_copy(ones_v, dsrc_sh.at[srcall.at[j]], sem, add=True)
            pltpu.async_copy(ones_v, ddst_sh.at[dstall.at[j]], sem, add=True)
        for u in range(DEG_U):
            pltpu.make_async_copy(ones_v, dsrc_sh.at[srcall.at[0]], sem).wait()
            pltpu.make_async_copy(ones_v, ddst_sh.at[dstall.at[0]], sem).wait()
        return carry

    lax.fori_loop(0, CHUNKS // DEG_U, body, 0)
    plsc.subcore_barrier()
    pltpu.sync_copy(dsrc_sh.at[pl.ds(s * DPT, DPT)], out_hbm.at[c, 0, pl.ds(s * DPT, DPT)])
    pltpu.sync_copy(ddst_sh.at[pl.ds(s * DPT, DPT)], out_hbm.at[c, 1, pl.ds(s * DPT, DPT)])


_deg = pl.kernel(
    _deg_body,
    out_type=jax.ShapeDtypeStruct((NC, 2, NPAD), jnp.float32),
    mesh=_mesh,
    scratch_types=[
        pltpu.VMEM((CHUNKS, K), jnp.int32),
        pltpu.VMEM((CHUNKS, K), jnp.int32),
        pltpu.VMEM((K,), jnp.float32),
        pltpu.VMEM((DPT,), jnp.float32),
        pltpu.VMEM_SHARED((NPAD,), jnp.float32),
        pltpu.VMEM_SHARED((NPAD,), jnp.float32),
        pltpu.SemaphoreType.DMA,
    ],
    compiler_params=_sc_params,
)


def _make_agg_body(C):
    ZR = RPT // 8

    def body(h_hbm, src2_hbm, dst2_hbm, out_hbm,
             srcall, dstall, rows_a, rows_b, acc_sh, gsem_a, gsem_b, isem):
        c = lax.axis_index("c")
        s = lax.axis_index("s")
        wid = s * NC + c
        pltpu.async_copy(src2_hbm.at[wid], srcall, gsem_b)
        pltpu.async_copy(dst2_hbm.at[wid], dstall, isem)

        def zfill(i, carry):
            r = i // (C // 16)
            k = i % (C // 16)
            rows_a[r, pl.ds(k * 16, 16)] = jnp.zeros((16,), jnp.float32)
            return carry

        lax.fori_loop(0, K * C // 16, zfill, 0)
        for t in range(8):
            pltpu.sync_copy(rows_a.at[pl.ds(0, ZR)],
                            acc_sh.at[pl.ds(s * RPT + t * ZR, ZR)])
        pltpu.make_async_copy(src2_hbm.at[wid], srcall, gsem_b).wait()
        pltpu.make_async_copy(dst2_hbm.at[wid], dstall, isem).wait()
        plsc.subcore_barrier()
        pltpu.async_copy(h_hbm.at[srcall.at[0]], rows_a, gsem_a)

        def body2(i, carry):
            ja = 2 * i
            jb = ja + 1
            pltpu.async_copy(h_hbm.at[srcall.at[jb]], rows_b, gsem_b)
            pltpu.make_async_copy(h_hbm.at[srcall.at[ja]], rows_a, gsem_a).wait()
            pltpu.sync_copy(rows_a, acc_sh.at[dstall.at[ja]], add=True)
            pltpu.async_copy(h_hbm.at[srcall.at[ja + 2]], rows_a, gsem_a)
            pltpu.make_async_copy(h_hbm.at[srcall.at[jb]], rows_b, gsem_b).wait()
            pltpu.sync_copy(rows_b, acc_sh.at[dstall.at[jb]], add=True)
            return carry

        lax.fori_loop(0, (CHUNKS - 1) // 2, body2, 0)
        pltpu.make_async_copy(h_hbm.at[srcall.at[CHUNKS - 1]], rows_a, gsem_a).wait()
        pltpu.sync_copy(rows_a, acc_sh.at[dstall.at[CHUNKS - 1]], add=True)
        plsc.subcore_barrier()
        pltpu.sync_copy(acc_sh.at[pl.ds(s * RPT, RPT)], out_hbm.at[c, pl.ds(s * RPT, RPT)])

    return body


def _make_agg(C):
    return pl.kernel(
        _make_agg_body(C),
        out_type=jax.ShapeDtypeStruct((NC, NR, C), jnp.float32),
        mesh=_mesh,
        scratch_types=[
            pltpu.VMEM((CHUNKS, K), jnp.int32),
            pltpu.VMEM((CHUNKS, K), jnp.int32),
            pltpu.VMEM((K, C), jnp.float32),
            pltpu.VMEM((K, C), jnp.float32),
            pltpu.VMEM_SHARED((NR, C), jnp.float32),
            pltpu.SemaphoreType.DMA,
            pltpu.SemaphoreType.DMA,
            pltpu.SemaphoreType.DMA,
        ],
        compiler_params=_sc_params,
    )


NBUF = 5


def _make_agg_ring(C):
    ZR = RPT // 8

    def body(h_hbm, src2_hbm, dst2_hbm, out_hbm, srcall, dstall,
             r0, r1, r2, r3, r4, g0, g1, g2, g3, g4,
             s0, s1, s2, s3, s4, isem, acc_sh):
        rows = [r0, r1, r2, r3, r4]
        gsems = [g0, g1, g2, g3, g4]
        ssems = [s0, s1, s2, s3, s4]
        c = lax.axis_index("c")
        s = lax.axis_index("s")
        wid = s * NC + c
        pltpu.async_copy(src2_hbm.at[wid], srcall, isem)
        pltpu.async_copy(dst2_hbm.at[wid], dstall, isem)

        def zfill(i, carry):
            r = i // (C // 16)
            k = i % (C // 16)
            r0[r, pl.ds(k * 16, 16)] = jnp.zeros((16,), jnp.float32)
            return carry

        lax.fori_loop(0, K * C // 16, zfill, 0)
        for t in range(8):
            pltpu.sync_copy(r0.at[pl.ds(0, ZR)],
                            acc_sh.at[pl.ds(s * RPT + t * ZR, ZR)])
        pltpu.make_async_copy(src2_hbm.at[wid], srcall, isem).wait()
        pltpu.make_async_copy(dst2_hbm.at[wid], dstall, isem).wait()
        plsc.subcore_barrier()
        for u in range(NBUF):
            pltpu.async_copy(h_hbm.at[srcall.at[u]], rows[u], gsems[u])

        def ring(t, carry):
            for u in range(NBUF):
                j = t * NBUF + u
                pltpu.make_async_copy(h_hbm.at[srcall.at[j]], rows[u], gsems[u]).wait()
                pltpu.async_copy(rows[u], acc_sh.at[dstall.at[j]], ssems[u], add=True)

                @pl.when(j + NBUF < CHUNKS)
                def _():
                    pltpu.make_async_copy(rows[u], acc_sh.at[dstall.at[0]], ssems[u]).wait()
                    pltpu.async_copy(h_hbm.at[srcall.at[j + NBUF]], rows[u], gsems[u])

            return carry

        lax.fori_loop(0, CHUNKS // NBUF, ring, 0)
        for u in range(NBUF):
            pltpu.make_async_copy(rows[u], acc_sh.at[dstall.at[0]], ssems[u]).wait()
        plsc.subcore_barrier()
        pltpu.sync_copy(acc_sh.at[pl.ds(s * RPT, RPT)], out_hbm.at[c, pl.ds(s * RPT, RPT)])

    return pl.kernel(
        body,
        out_type=jax.ShapeDtypeStruct((NC, NR, C), jnp.float32),
        mesh=_mesh,
        scratch_types=(
            [pltpu.VMEM((CHUNKS, K), jnp.int32)] * 2
            + [pltpu.VMEM((K, C), jnp.float32)] * NBUF
            + [pltpu.SemaphoreType.DMA] * (2 * NBUF + 1)
            + [pltpu.VMEM_SHARED((NR, C), jnp.float32)]
        ),
        compiler_params=_sc_params,
    )


NB_H = 3
RPT_H = N // NS


def _make_agg_ring_h(C):
    RING_T = (CHUNKS - 2) // NB_H

    def body(h_hbm, src2_hbm, dst2_hbm, out_hbm, srcall, dstall,
             r0, r1, r2, g0, g1, g2, s0, s1, s2, isem, acc_sh):
        rows = [r0, r1, r2]
        gsems = [g0, g1, g2]
        ssems = [s0, s1, s2]
        c = lax.axis_index("c")
        s = lax.axis_index("s")
        wid = s * NC + c
        pltpu.async_copy(src2_hbm.at[wid], srcall, isem)
        pltpu.async_copy(dst2_hbm.at[wid], dstall, isem)

        def zfill(i, carry):
            r = i // (C // 16)
            k = i % (C // 16)
            r0[r, pl.ds(k * 16, 16)] = jnp.zeros((16,), jnp.float32)
            return carry

        lax.fori_loop(0, K * C // 16, zfill, 0)
        for t in range(25):
            pltpu.sync_copy(r0.at[pl.ds(0, 25)],
                            acc_sh.at[pl.ds(s * RPT_H + t * 25, 25)])
        pltpu.make_async_copy(src2_hbm.at[wid], srcall, isem).wait()
        pltpu.make_async_copy(dst2_hbm.at[wid], dstall, isem).wait()
        plsc.subcore_barrier()
        for u in range(NB_H):
            pltpu.async_copy(h_hbm.at[srcall.at[u]], rows[u], gsems[u])

        def ring(t, carry):
            for u in range(NB_H):
                j = t * NB_H + u
                pltpu.make_async_copy(h_hbm.at[srcall.at[j]], rows[u], gsems[u]).wait()
                pltpu.async_copy(rows[u], acc_sh.at[dstall.at[j]], ssems[u], add=True)

                @pl.when(j + NB_H < CHUNKS)
                def _():
                    pltpu.make_async_copy(rows[u], acc_sh.at[dstall.at[0]], ssems[u]).wait()
                    pltpu.async_copy(h_hbm.at[srcall.at[j + NB_H]], rows[u], gsems[u])

            return carry

        lax.fori_loop(0, RING_T, ring, 0)
        for j, u in ((CHUNKS - 2, 0), (CHUNKS - 1, 1)):
            pltpu.make_async_copy(h_hbm.at[srcall.at[j]], rows[u], gsems[u]).wait()
            pltpu.async_copy(rows[u], acc_sh.at[dstall.at[j]], ssems[u], add=True)
        for u in range(NB_H):
            pltpu.make_async_copy(rows[u], acc_sh.at[dstall.at[0]], ssems[u]).wait()
        plsc.subcore_barrier()
        pltpu.sync_copy(acc_sh.at[pl.ds(s * RPT_H, RPT_H)],
                        out_hbm.at[c, pl.ds(s * RPT_H, RPT_H)])

    return pl.kernel(
        body,
        out_type=jax.ShapeDtypeStruct((NC, N, C), jnp.float32),
        mesh=_mesh,
        scratch_types=(
            [pltpu.VMEM((CHUNKS, K), jnp.int32)] * 2
            + [pltpu.VMEM((K, C), jnp.float32)] * NB_H
            + [pltpu.SemaphoreType.DMA] * (2 * NB_H + 1)
            + [pltpu.VMEM_SHARED((N, C), jnp.float32)]
        ),
        compiler_params=_sc_params,
    )


_agg_h = _make_agg_ring_h(H)
_agg_o = _make_agg_ring(O)

BN = 1000


def _tcb_body(x_ref, w_ref, degp_ref, h1_ref, ns_ref, nd_ref):
    degs = degp_ref[0, 0] + degp_ref[1, 0]
    degd = degp_ref[0, 1] + degp_ref[1, 1]
    ns = lax.rsqrt(jnp.maximum(degs, 1.0))
    nd = lax.rsqrt(jnp.maximum(degd, 1.0))
    ns_ref[...] = ns
    nd_ref[...] = nd
    h1_ref[...] = jnp.dot(x_ref[...], w_ref[...],
                          preferred_element_type=jnp.float32) * ns


_tcb = pl.pallas_call(
    _tcb_body,
    grid=(N // BN,),
    in_specs=[
        pl.BlockSpec((BN, F), lambda i: (i, 0)),
        pl.BlockSpec((F, H), lambda i: (0, 0)),
        pl.BlockSpec((NC, 2, BN, 1), lambda i: (0, 0, i, 0)),
    ],
    out_specs=[
        pl.BlockSpec((BN, H), lambda i: (i, 0)),
        pl.BlockSpec((BN, 1), lambda i: (i, 0)),
        pl.BlockSpec((BN, 1), lambda i: (i, 0)),
    ],
    out_shape=[
        jax.ShapeDtypeStruct((N, H), jnp.float32),
        jax.ShapeDtypeStruct((N, 1), jnp.float32),
        jax.ShapeDtypeStruct((N, 1), jnp.float32),
    ],
)


def _tc2_body(aggp_ref, nd_ref, b1_ref, w2_ref, ns_ref, h2_ref):
    h = (aggp_ref[0] + aggp_ref[1]) * nd_ref[...] + b1_ref[...]
    h = jnp.maximum(h, 0.0)
    h2_ref[...] = jnp.dot(h, w2_ref[...],
                          preferred_element_type=jnp.float32) * ns_ref[...]


_tc2 = pl.pallas_call(
    _tc2_body,
    grid=(N // BN,),
    in_specs=[
        pl.BlockSpec((NC, BN, H), lambda i: (0, i, 0)),
        pl.BlockSpec((BN, 1), lambda i: (i, 0)),
        pl.BlockSpec((1, H), lambda i: (0, 0)),
        pl.BlockSpec((H, O), lambda i: (0, 0)),
        pl.BlockSpec((BN, 1), lambda i: (i, 0)),
    ],
    out_specs=pl.BlockSpec((BN, O), lambda i: (i, 0)),
    out_shape=jax.ShapeDtypeStruct((N, O), jnp.float32),
)


def _tc3_body(aggp_ref, nd_ref, b2_ref, o_ref):
    o_ref[...] = jnp.maximum(
        (aggp_ref[0] + aggp_ref[1]) * nd_ref[...] + b2_ref[...], 0.0)


_tc3 = pl.pallas_call(
    _tc3_body,
    grid=(N // BN,),
    in_specs=[
        pl.BlockSpec((NC, BN, O), lambda i: (0, i, 0)),
        pl.BlockSpec((BN, 1), lambda i: (i, 0)),
        pl.BlockSpec((1, O), lambda i: (0, 0)),
    ],
    out_specs=pl.BlockSpec((BN, O), lambda i: (i, 0)),
    out_shape=jax.ShapeDtypeStruct((N, O), jnp.float32),
)


def kernel(graph, node_input, W1, b1, W2, b2):
    src2 = graph[0].reshape(NW, CHUNKS, K)
    dst2 = graph[1].reshape(NW, CHUNKS, K)

    degp = _deg(src2, dst2)
    degp4 = degp.reshape(NC, 2, NPAD, 1)
    h1, ns, nd = _tcb(node_input, W1, degp4)
    aggp1 = _agg_h(h1, src2, dst2)
    h2 = _tc2(aggp1, nd, b1.reshape(1, H), W2, ns)
    aggp2 = _agg_o(h2, src2, dst2)
    return _tc3(aggp2, nd, b2.reshape(1, O))


# speedup vs baseline: 1.0013x; 1.0013x over previous
import jax
import jax.numpy as jnp
from jax import lax
from jax.experimental import pallas as pl
from jax.experimental.pallas import tpu as pltpu
from jax.experimental.pallas import tpu_sc as plsc

N = 10000
E = 320000
F = 128
H = 128
O = 64

NC = 2
NS = 16
NW = NC * NS
EPT = E // NW
K = 80
CHUNKS = EPT // K
NPAD = 10240
RPT = 632
NR = NS * RPT
DPT = NPAD // NS

_mesh = plsc.VectorSubcoreMesh(core_axis_name="c", subcore_axis_name="s")
_sc_params = pltpu.CompilerParams(use_tc_tiling_on_sc=False)


DEG_U = 5


def _deg_body(src2_hbm, dst2_hbm, out_hbm,
              srcall, dstall, ones_v, zv, dsrc_sh, ddst_sh, sem):
    c = lax.axis_index("c")
    s = lax.axis_index("s")
    wid = s * NC + c
    pltpu.async_copy(src2_hbm.at[wid], srcall, sem)
    pltpu.async_copy(dst2_hbm.at[wid], dstall, sem)
    for i in range(K // 16):
        ones_v[pl.ds(i * 16, 16)] = jnp.ones((16,), jnp.float32)

    def zfill(i, carry):
        zv[pl.ds(i * 16, 16)] = jnp.zeros((16,), jnp.float32)
        return carry

    lax.fori_loop(0, DPT // 16, zfill, 0)
    pltpu.sync_copy(zv, dsrc_sh.at[pl.ds(s * DPT, DPT)])
    pltpu.sync_copy(zv, ddst_sh.at[pl.ds(s * DPT, DPT)])
    pltpu.make_async_copy(src2_hbm.at[wid], srcall, sem).wait()
    pltpu.make_async_copy(dst2_hbm.at[wid], dstall, sem).wait()
    plsc.subcore_barrier()

    def body(t, carry):
        for u in range(DEG_U):
            j = t * DEG_U + u
            pltpu.async_copy(ones_v, dsrc_sh.at[srcall.at[j]], sem, add=True)
            pltpu.async_copy(ones_v, ddst_sh.at[dstall.at[j]], sem, add=True)
        for u in range(DEG_U):
            pltpu.make_async_copy(ones_v, dsrc_sh.at[srcall.at[0]], sem).wait()
            pltpu.make_async_copy(ones_v, ddst_sh.at[dstall.at[0]], sem).wait()
        return carry

    lax.fori_loop(0, CHUNKS // DEG_U, body, 0)
    plsc.subcore_barrier()
    pltpu.sync_copy(dsrc_sh.at[pl.ds(s * DPT, DPT)], out_hbm.at[c, 0, pl.ds(s * DPT, DPT)])
    pltpu.sync_copy(ddst_sh.at[pl.ds(s * DPT, DPT)], out_hbm.at[c, 1, pl.ds(s * DPT, DPT)])


_deg = pl.kernel(
    _deg_body,
    out_type=jax.ShapeDtypeStruct((NC, 2, NPAD), jnp.float32),
    mesh=_mesh,
    scratch_types=[
        pltpu.VMEM((CHUNKS, K), jnp.int32),
        pltpu.VMEM((CHUNKS, K), jnp.int32),
        pltpu.VMEM((K,), jnp.float32),
        pltpu.VMEM((DPT,), jnp.float32),
        pltpu.VMEM_SHARED((NPAD,), jnp.float32),
        pltpu.VMEM_SHARED((NPAD,), jnp.float32),
        pltpu.SemaphoreType.DMA,
    ],
    compiler_params=_sc_params,
)


NBUF = 5


def _make_agg_ring(C):
    ZR = RPT // 8

    def body(h_hbm, src2_hbm, dst2_hbm, out_hbm, srcall, dstall,
             r0, r1, r2, r3, r4, g0, g1, g2, g3, g4,
             s0, s1, s2, s3, s4, isem, acc_sh):
        rows = [r0, r1, r2, r3, r4]
        gsems = [g0, g1, g2, g3, g4]
        ssems = [s0, s1, s2, s3, s4]
        c = lax.axis_index("c")
        s = lax.axis_index("s")
        wid = s * NC + c
        pltpu.async_copy(src2_hbm.at[wid], srcall, isem)
        pltpu.async_copy(dst2_hbm.at[wid], dstall, isem)

        def zfill(i, carry):
            r = i // (C // 16)
            k = i % (C // 16)
            r0[r, pl.ds(k * 16, 16)] = jnp.zeros((16,), jnp.float32)
            return carry

        lax.fori_loop(0, K * C // 16, zfill, 0)
        for t in range(8):
            pltpu.sync_copy(r0.at[pl.ds(0, ZR)],
                            acc_sh.at[pl.ds(s * RPT + t * ZR, ZR)])
        pltpu.make_async_copy(src2_hbm.at[wid], srcall, isem).wait()
        pltpu.make_async_copy(dst2_hbm.at[wid], dstall, isem).wait()
        plsc.subcore_barrier()
        for u in range(NBUF):
            pltpu.async_copy(h_hbm.at[srcall.at[u]], rows[u], gsems[u])

        def ring(t, carry):
            for u in range(NBUF):
                j = t * NBUF + u
                pltpu.make_async_copy(h_hbm.at[srcall.at[j]], rows[u], gsems[u]).wait()
                pltpu.async_copy(rows[u], acc_sh.at[dstall.at[j]], ssems[u], add=True)

                @pl.when(j + NBUF < CHUNKS)
                def _():
                    pltpu.make_async_copy(rows[u], acc_sh.at[dstall.at[0]], ssems[u]).wait()
                    pltpu.async_copy(h_hbm.at[srcall.at[j + NBUF]], rows[u], gsems[u])

            return carry

        lax.fori_loop(0, CHUNKS // NBUF, ring, 0)
        for u in range(NBUF):
            pltpu.make_async_copy(rows[u], acc_sh.at[dstall.at[0]], ssems[u]).wait()
        plsc.subcore_barrier()
        pltpu.sync_copy(acc_sh.at[pl.ds(s * RPT, RPT)], out_hbm.at[c, pl.ds(s * RPT, RPT)])

    return pl.kernel(
        body,
        out_type=jax.ShapeDtypeStruct((NC, NR, C), jnp.float32),
        mesh=_mesh,
        scratch_types=(
            [pltpu.VMEM((CHUNKS, K), jnp.int32)] * 2
            + [pltpu.VMEM((K, C), jnp.float32)] * NBUF
            + [pltpu.SemaphoreType.DMA] * (2 * NBUF + 1)
            + [pltpu.VMEM_SHARED((NR, C), jnp.float32)]
        ),
        compiler_params=_sc_params,
    )


NB_H = 3
RPT_H = N // NS


def _make_agg_ring_h(C):
    RING_T = (CHUNKS - 2) // NB_H

    def body(h_hbm, src2_hbm, dst2_hbm, out_hbm, srcall, dstall,
             r0, r1, r2, g0, g1, g2, s0, s1, s2, isem, acc_sh):
        rows = [r0, r1, r2]
        gsems = [g0, g1, g2]
        ssems = [s0, s1, s2]
        c = lax.axis_index("c")
        s = lax.axis_index("s")
        wid = s * NC + c
        pltpu.async_copy(src2_hbm.at[wid], srcall, isem)
        pltpu.async_copy(dst2_hbm.at[wid], dstall, isem)

        def zfill(i, carry):
            r = i // (C // 16)
            k = i % (C // 16)
            r0[r, pl.ds(k * 16, 16)] = jnp.zeros((16,), jnp.float32)
            return carry

        lax.fori_loop(0, K * C // 16, zfill, 0)
        for t in range(25):
            pltpu.sync_copy(r0.at[pl.ds(0, 25)],
                            acc_sh.at[pl.ds(s * RPT_H + t * 25, 25)])
        pltpu.make_async_copy(src2_hbm.at[wid], srcall, isem).wait()
        pltpu.make_async_copy(dst2_hbm.at[wid], dstall, isem).wait()
        plsc.subcore_barrier()
        for u in range(NB_H):
            pltpu.async_copy(h_hbm.at[srcall.at[u]], rows[u], gsems[u])

        def ring(t, carry):
            for u in range(NB_H):
                j = t * NB_H + u
                pltpu.make_async_copy(h_hbm.at[srcall.at[j]], rows[u], gsems[u]).wait()
                pltpu.async_copy(rows[u], acc_sh.at[dstall.at[j]], ssems[u], add=True)

                @pl.when(j + NB_H < CHUNKS)
                def _():
                    pltpu.make_async_copy(rows[u], acc_sh.at[dstall.at[0]], ssems[u]).wait()
                    pltpu.async_copy(h_hbm.at[srcall.at[j + NB_H]], rows[u], gsems[u])

            return carry

        lax.fori_loop(0, RING_T, ring, 0)
        for j, u in ((CHUNKS - 2, 0), (CHUNKS - 1, 1)):
            pltpu.make_async_copy(h_hbm.at[srcall.at[j]], rows[u], gsems[u]).wait()
            pltpu.async_copy(rows[u], acc_sh.at[dstall.at[j]], ssems[u], add=True)
        for u in range(NB_H):
            pltpu.make_async_copy(rows[u], acc_sh.at[dstall.at[0]], ssems[u]).wait()
        plsc.subcore_barrier()
        pltpu.sync_copy(acc_sh.at[pl.ds(s * RPT_H, RPT_H)],
                        out_hbm.at[c, pl.ds(s * RPT_H, RPT_H)])

    return pl.kernel(
        body,
        out_type=jax.ShapeDtypeStruct((NC, N, C), jnp.float32),
        mesh=_mesh,
        scratch_types=(
            [pltpu.VMEM((CHUNKS, K), jnp.int32)] * 2
            + [pltpu.VMEM((K, C), jnp.float32)] * NB_H
            + [pltpu.SemaphoreType.DMA] * (2 * NB_H + 1)
            + [pltpu.VMEM_SHARED((N, C), jnp.float32)]
        ),
        compiler_params=_sc_params,
    )


_agg_h = _make_agg_ring_h(H)
_agg_o = _make_agg_ring(O)

BN = 1000


def _tcb_body(x_ref, w_ref, degp_ref, h1_ref, ns_ref, nd_ref):
    degs = degp_ref[0, 0] + degp_ref[1, 0]
    degd = degp_ref[0, 1] + degp_ref[1, 1]
    ns = lax.rsqrt(jnp.maximum(degs, 1.0))
    nd = lax.rsqrt(jnp.maximum(degd, 1.0))
    ns_ref[...] = ns
    nd_ref[...] = nd
    h1_ref[...] = jnp.dot(x_ref[...], w_ref[...],
                          preferred_element_type=jnp.float32) * ns


_tcb = pl.pallas_call(
    _tcb_body,
    grid=(N // BN,),
    in_specs=[
        pl.BlockSpec((BN, F), lambda i: (i, 0)),
        pl.BlockSpec((F, H), lambda i: (0, 0)),
        pl.BlockSpec((NC, 2, BN, 1), lambda i: (0, 0, i, 0)),
    ],
    out_specs=[
        pl.BlockSpec((BN, H), lambda i: (i, 0)),
        pl.BlockSpec((BN, 1), lambda i: (i, 0)),
        pl.BlockSpec((BN, 1), lambda i: (i, 0)),
    ],
    out_shape=[
        jax.ShapeDtypeStruct((N, H), jnp.float32),
        jax.ShapeDtypeStruct((N, 1), jnp.float32),
        jax.ShapeDtypeStruct((N, 1), jnp.float32),
    ],
)


def _tc2_body(aggp_ref, nd_ref, b1_ref, w2_ref, ns_ref, h2_ref):
    h = (aggp_ref[0] + aggp_ref[1]) * nd_ref[...] + b1_ref[...]
    h = jnp.maximum(h, 0.0)
    h2_ref[...] = jnp.dot(h, w2_ref[...],
                          preferred_element_type=jnp.float32) * ns_ref[...]


_tc2 = pl.pallas_call(
    _tc2_body,
    grid=(N // BN,),
    in_specs=[
        pl.BlockSpec((NC, BN, H), lambda i: (0, i, 0)),
        pl.BlockSpec((BN, 1), lambda i: (i, 0)),
        pl.BlockSpec((1, H), lambda i: (0, 0)),
        pl.BlockSpec((H, O), lambda i: (0, 0)),
        pl.BlockSpec((BN, 1), lambda i: (i, 0)),
    ],
    out_specs=pl.BlockSpec((BN, O), lambda i: (i, 0)),
    out_shape=jax.ShapeDtypeStruct((N, O), jnp.float32),
)


def _tc3_body(aggp_ref, nd_ref, b2_ref, o_ref):
    o_ref[...] = jnp.maximum(
        (aggp_ref[0] + aggp_ref[1]) * nd_ref[...] + b2_ref[...], 0.0)


_tc3 = pl.pallas_call(
    _tc3_body,
    grid=(N // BN,),
    in_specs=[
        pl.BlockSpec((NC, BN, O), lambda i: (0, i, 0)),
        pl.BlockSpec((BN, 1), lambda i: (i, 0)),
        pl.BlockSpec((1, O), lambda i: (0, 0)),
    ],
    out_specs=pl.BlockSpec((BN, O), lambda i: (i, 0)),
    out_shape=jax.ShapeDtypeStruct((N, O), jnp.float32),
)


def kernel(graph, node_input, W1, b1, W2, b2):
    src2 = graph[0].reshape(NW, CHUNKS, K)
    dst2 = graph[1].reshape(NW, CHUNKS, K)

    degp = _deg(src2, dst2)
    degp4 = degp.reshape(NC, 2, NPAD, 1)
    h1, ns, nd = _tcb(node_input, W1, degp4)
    aggp1 = _agg_h(h1, src2, dst2)
    h2 = _tc2(aggp1, nd, b1.reshape(1, H), W2, ns)
    aggp2 = _agg_o(h2, src2, dst2)
    return _tc3(aggp2, nd, b2.reshape(1, O))
